# baseline (speedup 1.0000x reference)
import jax
import jax.numpy as jnp
from jax.experimental import pallas as pl
from jax.experimental.pallas import tpu as pltpu


def _mm_bf16(a, b):
    return jax.lax.dot_general(a, b, (((1,), (0,)), ((), ())),
                               preferred_element_type=jnp.float32)


def _matmul_ta_bf16(a, b):
    return jax.lax.dot_general(a, b, (((0,), (0,)), ((), ())),
                               preferred_element_type=jnp.float32)


def _split(v):
    hi = v.astype(jnp.bfloat16)
    lo = (v - hi.astype(jnp.float32)).astype(jnp.bfloat16)
    return hi, lo


def _matmul3(h, w):
    h1, h2 = _split(h)
    w1, w2 = w
    return _mm_bf16(h1, w1) + (_mm_bf16(h1, w2) + _mm_bf16(h2, w1))


def _body(x_ref, M_ref, W1_ref, b1_ref, W2_ref, b2_ref, W3_ref, b3_ref,
          Wg1_ref, bg1_ref, Wg2_ref, bg2_ref, Wg3_ref, bg3_ref,
          Wg4_ref, bg4_ref, Wg5_ref, bg5_ref, Wg6_ref, bg6_ref, out_ref,
          m_vmem, sem):
    n = m_vmem.shape[0]
    copy = pltpu.make_async_copy(M_ref, m_vmem, sem)
    copy.start()

    W1 = _split(W1_ref[...])
    W2 = _split(W2_ref[...])
    W3 = _split(W3_ref[...])
    Wg1 = _split(Wg1_ref[...])
    Wg2 = _split(Wg2_ref[...])
    Wg3 = _split(Wg3_ref[...])
    Wg4 = _split(Wg4_ref[...])
    Wg5 = _split(Wg5_ref[...])
    Wg6 = _split(Wg6_ref[...])

    x = x_ref[...]
    x1l = _matmul3(x, W1) + b1_ref[...]
    hw1 = _matmul3(x1l, Wg1)

    copy.wait()
    M = m_vmem[...]
    ones = jnp.ones((n, 1), jnp.bfloat16)
    deg = _matmul_ta_bf16(M, ones) + 1.0
    dinv = jax.lax.rsqrt(deg)
    dinv_h = 0.5 * dinv
    dinv_q = 0.25 * dinv

    def gcn(h, w, bb, dscale, bscale):
        return nprop(_matmul3(h, w), bb, dscale, bscale)

    def nprop(hw, bb, dscale, bscale):
        g = hw * dinv
        gb = g.astype(jnp.bfloat16)
        t = _matmul_ta_bf16(M, gb)
        return (t + g) * dscale + bscale * bb

    x1 = x1l + jax.nn.relu(nprop(hw1, bg1_ref[...], dinv, 1.0))
    x2l = _matmul3(x1, W2) + b2_ref[...]
    x2 = x2l + jax.nn.relu(gcn(x2l, Wg2, bg2_ref[...], dinv, 1.0))
    x3l = _matmul3(x2, W3) + b3_ref[...]
    x3 = x3l + jax.nn.relu(gcn(x3l, Wg3, bg3_ref[...], dinv_h, 0.5))
    x4 = x3 + jax.nn.relu(gcn(x3, Wg4, bg4_ref[...], dinv_h, 0.5))
    x5 = x4 + jax.nn.relu(gcn(x4, Wg5, bg5_ref[...], dinv_q, 0.25))
    out_ref[...] = x5 + gcn(x5, Wg6, bg6_ref[...], dinv_q, 0.25)


def kernel(x, A_neg, A_pos, W1, b1, W2, b2, W3, b3, Wg1, bg1, Wg2, bg2,
           Wg3, bg3, Wg4, bg4, Wg5, bg5, Wg6, bg6):
    del A_pos
    n, dout = x.shape[0], Wg3.shape[0]
    Mbf = (A_neg != 0).astype(jnp.bfloat16)
    row = lambda v: v.reshape(1, -1)
    vmem = pl.BlockSpec(memory_space=pltpu.MemorySpace.VMEM)
    specs = [vmem, pl.BlockSpec(memory_space=pltpu.MemorySpace.HBM)]
    specs += [vmem] * 18
    return pl.pallas_call(
        _body,
        in_specs=specs,
        out_specs=vmem,
        scratch_shapes=[pltpu.VMEM((n, n), jnp.bfloat16),
                        pltpu.SemaphoreType.DMA],
        out_shape=jax.ShapeDtypeStruct((n, dout), jnp.float32),
    )(x, Mbf, W1, row(b1), W2, row(b2), W3, row(b3),
      Wg1, row(bg1), Wg2, row(bg2), Wg3, row(bg3),
      Wg4, row(bg4), Wg5, row(bg5), Wg6, row(bg6))


# speedup vs baseline: 1.2112x; 1.2112x over previous
import jax
import jax.numpy as jnp
from jax.experimental import pallas as pl


def _mm_bf16(a, b):
    return jax.lax.dot_general(a, b, (((1,), (0,)), ((), ())),
                               preferred_element_type=jnp.float32)


def _matmul_ta_bf16(a, b):
    return jax.lax.dot_general(a, b, (((0,), (0,)), ((), ())),
                               preferred_element_type=jnp.float32)


def _split(v):
    hi = v.astype(jnp.bfloat16)
    lo = (v - hi.astype(jnp.float32)).astype(jnp.bfloat16)
    return hi, lo


def _matmul3(h, w):
    h1, h2 = _split(h)
    w1, w2 = w
    return _mm_bf16(h1, w1) + (_mm_bf16(h1, w2) + _mm_bf16(h2, w1))


def _body(x_ref, A_ref, W1_ref, b1_ref, W2_ref, b2_ref, W3_ref, b3_ref,
          Wg1_ref, bg1_ref, Wg2_ref, bg2_ref, Wg3_ref, bg3_ref,
          Wg4_ref, bg4_ref, Wg5_ref, bg5_ref, Wg6_ref, bg6_ref, out_ref):
    n = A_ref.shape[0]
    M = (A_ref[...] != 0).astype(jnp.bfloat16)
    ones = jnp.ones((n, 1), jnp.bfloat16)
    deg = _matmul_ta_bf16(M, ones) + 1.0
    dinv = jax.lax.rsqrt(deg)
    dinv_h = 0.5 * dinv
    dinv_q = 0.25 * dinv

    W1 = _split(W1_ref[...])
    W2 = _split(W2_ref[...])
    W3 = _split(W3_ref[...])
    Wg1 = _split(Wg1_ref[...])
    Wg2 = _split(Wg2_ref[...])
    Wg3 = _split(Wg3_ref[...])
    Wg4 = _split(Wg4_ref[...])
    Wg5 = _split(Wg5_ref[...])
    Wg6 = _split(Wg6_ref[...])

    def gcn(h, w, bb, dscale, bscale):
        g = _matmul3(h, w) * dinv
        gb = g.astype(jnp.bfloat16)
        t = _matmul_ta_bf16(M, gb)
        return (t + g) * dscale + bscale * bb

    x = x_ref[...]
    x1l = _matmul3(x, W1) + b1_ref[...]
    x1 = x1l + jax.nn.relu(gcn(x1l, Wg1, bg1_ref[...], dinv, 1.0))
    x2l = _matmul3(x1, W2) + b2_ref[...]
    x2 = x2l + jax.nn.relu(gcn(x2l, Wg2, bg2_ref[...], dinv, 1.0))
    x3l = _matmul3(x2, W3) + b3_ref[...]
    x3 = x3l + jax.nn.relu(gcn(x3l, Wg3, bg3_ref[...], dinv_h, 0.5))
    x4 = x3 + jax.nn.relu(gcn(x3, Wg4, bg4_ref[...], dinv_h, 0.5))
    x5 = x4 + jax.nn.relu(gcn(x4, Wg5, bg5_ref[...], dinv_q, 0.25))
    out_ref[...] = x5 + gcn(x5, Wg6, bg6_ref[...], dinv_q, 0.25)


def kernel(x, A_neg, A_pos, W1, b1, W2, b2, W3, b3, Wg1, bg1, Wg2, bg2,
           Wg3, bg3, Wg4, bg4, Wg5, bg5, Wg6, bg6):
    del A_pos
    n, dout = x.shape[0], Wg3.shape[0]
    row = lambda v: v.reshape(1, -1)
    return pl.pallas_call(
        _body,
        out_shape=jax.ShapeDtypeStruct((n, dout), jnp.float32),
    )(x, A_neg, W1, row(b1), W2, row(b2), W3, row(b3),
      Wg1, row(bg1), Wg2, row(bg2), Wg3, row(bg3),
      Wg4, row(bg4), Wg5, row(bg5), Wg6, row(bg6))
